# baseline (speedup 1.0000x reference)
import jax
import jax.numpy as jnp
from jax.experimental import pallas as pl
from jax.experimental.pallas import tpu as pltpu

_B = 8
_N = 2048
_TQ = 128
_NT = _N // _TQ
_LANES = 128
_NSL = _N // _LANES
_BIG = 1e30
_F32 = jnp.float32
_BF16 = jnp.bfloat16


def _split3(x):
    hi = x.astype(_BF16).astype(_F32)
    r = x - hi
    mid = r.astype(_BF16).astype(_F32)
    lo = (r - mid).astype(_BF16).astype(_F32)
    return jnp.concatenate([hi, mid, lo], axis=1).astype(_BF16)


def _lane_top2(dist):
    m0 = dist[:, 0:_LANES]
    m1 = jnp.full_like(m0, _BIG)
    for k in range(1, _NSL):
        x = dist[:, k * _LANES:(k + 1) * _LANES]
        lo = jnp.minimum(m0, x)
        hi = jnp.maximum(m0, x)
        m1 = jnp.minimum(m1, hi)
        m0 = lo
    return m0, m1


def _top3_vals(dist):
    m0, m1 = _lane_top2(dist)
    mm0 = jnp.min(m0, axis=1, keepdims=True)
    a0 = jnp.where(m0 == mm0, _BIG, m0)
    a1 = jnp.where(m1 == mm0, _BIG, m1)
    mm1 = jnp.min(jnp.minimum(a0, a1), axis=1, keepdims=True)
    b0 = jnp.where(a0 == mm1, _BIG, a0)
    b1 = jnp.where(a1 == mm1, _BIG, a1)
    mm2 = jnp.min(jnp.minimum(b0, b1), axis=1, keepdims=True)
    return mm1, mm2


def _self_kappa(dist, pts9, pts_t_tile, nrm_t_tile):
    mm1, mm2 = _top3_vals(dist)
    oh1 = (dist == mm1).astype(_BF16)
    oh2 = (dist == mm2).astype(_BF16)
    r1 = jax.lax.dot(oh1, pts9, preferred_element_type=_F32)
    r2 = jax.lax.dot(oh2, pts9, preferred_element_type=_F32)

    def term(r):
        c = (r[:, 0:3] + r[:, 3:6]) + r[:, 6:9]
        v = c - pts_t_tile
        nv = jnp.sqrt(jnp.sum(v * v, axis=1, keepdims=True))
        s = jnp.sum(v * nrm_t_tile, axis=1, keepdims=True)
        return jnp.abs(s / jnp.maximum(nv, 1e-12))

    return 0.5 * (term(r1) + term(r2))


def _curv_kernel(ori_ref, adv_ref, oriT_ref, advT_ref, nrmT_ref,
                 out_ref, kappa_ref, acc_ref, ori9_ref, adv9_ref, pay12_ref):
    b = pl.program_id(0)
    phase = pl.program_id(1)
    t = pl.program_id(2)
    sl = pl.ds(t * _TQ, _TQ)

    @pl.when(jnp.logical_and(phase == 0, t == 0))
    def _():
        ori9_ref[...] = _split3(oriT_ref[0])
        adv9_ref[...] = _split3(advT_ref[0])

    @pl.when(phase == 0)
    def _():
        p = ori_ref[0]
        pt_t = oriT_ref[0, sl, :]
        nt_t = nrmT_ref[0, sl, :]
        n2r = jnp.sum(p * p, axis=0, keepdims=True)
        n2c = jnp.sum(pt_t * pt_t, axis=1, keepdims=True)
        g2 = jax.lax.dot(pt_t * -2.0, p, preferred_element_type=_F32)
        dist = (n2c + n2r) + g2
        kappa_ref[sl, :] = _self_kappa(dist, ori9_ref[...], pt_t, nt_t)

    @pl.when(jnp.logical_and(phase == 1, t == 0))
    def _():
        pay12_ref[...] = jnp.concatenate(
            [_split3(nrmT_ref[0]), _split3(kappa_ref[...])], axis=1)

    @pl.when(phase == 1)
    def _():
        o = ori_ref[0]
        a = adv_ref[0]
        at_t = advT_ref[0, sl, :]
        n2o = jnp.sum(o * o, axis=0, keepdims=True)
        n2a_c = jnp.sum(at_t * at_t, axis=1, keepdims=True)
        at2 = at_t * -2.0
        g_ao = jax.lax.dot(at2, o, preferred_element_type=_F32)
        d_ao = (n2a_c + n2o) + g_ao
        cm = d_ao[:, 0:_LANES]
        for k in range(1, _NSL):
            cm = jnp.minimum(cm, d_ao[:, k * _LANES:(k + 1) * _LANES])
        mm = jnp.min(cm, axis=1, keepdims=True)
        ohf = (d_ao == mm).astype(_BF16)
        raw = jax.lax.dot(ohf, pay12_ref[...], preferred_element_type=_F32)
        nadv = (raw[:, 0:3] + raw[:, 3:6]) + raw[:, 6:9]
        onenn = (raw[:, 9:10] + raw[:, 10:11]) + raw[:, 11:12]
        n2a_r = jnp.sum(a * a, axis=0, keepdims=True)
        g_aa = jax.lax.dot(at2, a, preferred_element_type=_F32)
        d_aa = (n2a_c + n2a_r) + g_aa
        advk = _self_kappa(d_aa, adv9_ref[...], at_t, nadv)
        diff = advk - onenn
        part = jnp.reshape(jnp.sum(diff * diff), (1, 1))

        @pl.when(jnp.logical_and(b == 0, t == 0))
        def _():
            acc_ref[...] = jnp.zeros((1, 1), _F32)

        acc_ref[...] += part
        out_ref[...] = acc_ref[...] * (1.0 / (_B * _N))


def kernel(ori_data, adv_data, ori_normal):
    oriT = jnp.transpose(ori_data, (0, 2, 1))
    advT = jnp.transpose(adv_data, (0, 2, 1))
    nrmT = jnp.transpose(ori_normal, (0, 2, 1))
    row_spec = pl.BlockSpec((1, 3, _N), lambda b, p, t: (b, 0, 0))
    t_spec = pl.BlockSpec((1, _N, 3), lambda b, p, t: (b, 0, 0))
    out = pl.pallas_call(
        _curv_kernel,
        grid=(_B, 2, _NT),
        in_specs=[row_spec, row_spec, t_spec, t_spec, t_spec],
        out_specs=pl.BlockSpec((1, 1), lambda b, p, t: (0, 0)),
        out_shape=jax.ShapeDtypeStruct((1, 1), _F32),
        scratch_shapes=[
            pltpu.VMEM((_N, 1), _F32),
            pltpu.VMEM((1, 1), _F32),
            pltpu.VMEM((_N, 9), _BF16),
            pltpu.VMEM((_N, 9), _BF16),
            pltpu.VMEM((_N, 12), _BF16),
        ],
    )(ori_data, adv_data, oriT, advT, nrmT)
    return out[0, 0]


# speedup vs baseline: 1.4605x; 1.4605x over previous
import jax
import jax.numpy as jnp
from jax.experimental import pallas as pl
from jax.experimental.pallas import tpu as pltpu

_B = 8
_N = 2048
_TQ = 256
_NT = _N // _TQ
_SUB = 128
_NSL = _N // _SUB
_BIG = 1e30
_F32 = jnp.float32
_BF16 = jnp.bfloat16


def _split3_rows(x):
    hi = x.astype(_BF16).astype(_F32)
    r = x - hi
    mid = r.astype(_BF16).astype(_F32)
    lo = (r - mid).astype(_BF16).astype(_F32)
    return jnp.concatenate([hi, mid, lo], axis=0).astype(_BF16)


def _split3_cols_f32(x):
    hi = x.astype(_BF16).astype(_F32)
    r = x - hi
    mid = r.astype(_BF16).astype(_F32)
    lo = (r - mid).astype(_BF16).astype(_F32)
    return jnp.concatenate([hi, mid, lo], axis=1)


def _lane_top2_T(dist_t):
    m0 = dist_t[0:_SUB, :]
    m1 = jnp.full_like(m0, _BIG)
    for k in range(1, _NSL):
        x = dist_t[k * _SUB:(k + 1) * _SUB, :]
        lo = jnp.minimum(m0, x)
        hi = jnp.maximum(m0, x)
        m1 = jnp.minimum(m1, hi)
        m0 = lo
    return m0, m1


def _top3_vals_T(dist_t):
    m0, m1 = _lane_top2_T(dist_t)
    mm0 = jnp.min(m0, axis=0, keepdims=True)
    a0 = jnp.where(m0 == mm0, _BIG, m0)
    a1 = jnp.where(m1 == mm0, _BIG, m1)
    mm1 = jnp.min(jnp.minimum(a0, a1), axis=0, keepdims=True)
    b0 = jnp.where(a0 == mm1, _BIG, a0)
    b1 = jnp.where(a1 == mm1, _BIG, a1)
    mm2 = jnp.min(jnp.minimum(b0, b1), axis=0, keepdims=True)
    return mm1, mm2


def _self_kappa_T(dist_t, pts9_t, q_sl, nrm_sl):
    mm1, mm2 = _top3_vals_T(dist_t)
    oh1 = (dist_t == mm1).astype(_BF16)
    oh2 = (dist_t == mm2).astype(_BF16)
    r1 = jax.lax.dot(pts9_t, oh1, preferred_element_type=_F32)
    r2 = jax.lax.dot(pts9_t, oh2, preferred_element_type=_F32)

    def term(r):
        c = (r[0:3, :] + r[3:6, :]) + r[6:9, :]
        v = c - q_sl
        nv = jnp.sqrt(jnp.sum(v * v, axis=0, keepdims=True))
        s = jnp.sum(v * nrm_sl, axis=0, keepdims=True)
        return jnp.abs(s / jnp.maximum(nv, 1e-12))

    return 0.5 * (term(r1) + term(r2))


def _curv_kernel(ori_ref, adv_ref, nrm_ref, oriT_ref, advT_ref,
                 out_ref, kappa_ref, acc_ref,
                 lhso_ref, lhsa_ref, pts9o_ref, pts9a_ref, pay12_ref):
    b = pl.program_id(0)
    phase = pl.program_id(1)
    t = pl.program_id(2)
    sl = pl.ds(t * _TQ, _TQ)
    ones3 = jnp.ones((3, _TQ), _F32)

    @pl.when(jnp.logical_and(phase == 0, t == 0))
    def _():
        ot = oriT_ref[0]
        at = advT_ref[0]
        n2o = jnp.sum(ot * ot, axis=1, keepdims=True)
        n2a = jnp.sum(at * at, axis=1, keepdims=True)
        lhso_ref[...] = jnp.concatenate([ot, _split3_cols_f32(n2o)], axis=1)
        lhsa_ref[...] = jnp.concatenate([at, _split3_cols_f32(n2a)], axis=1)
        pts9o_ref[...] = _split3_rows(ori_ref[0])
        pts9a_ref[...] = _split3_rows(adv_ref[0])

    @pl.when(phase == 0)
    def _():
        o_sl = ori_ref[0, :, sl]
        n_sl = nrm_ref[0, :, sl]
        rhs = jnp.concatenate([o_sl * -2.0, ones3], axis=0)
        g = jax.lax.dot(lhso_ref[...], rhs, preferred_element_type=_F32)
        n2q = jnp.sum(o_sl * o_sl, axis=0, keepdims=True)
        dist_t = g + n2q
        kappa_ref[:, sl] = _self_kappa_T(dist_t, pts9o_ref[...], o_sl, n_sl)

    @pl.when(jnp.logical_and(phase == 1, t == 0))
    def _():
        pay12_ref[...] = jnp.concatenate(
            [_split3_rows(nrm_ref[0]), _split3_rows(kappa_ref[...])], axis=0)

    @pl.when(phase == 1)
    def _():
        a_sl = adv_ref[0, :, sl]
        rhs = jnp.concatenate([a_sl * -2.0, ones3], axis=0)
        n2q = jnp.sum(a_sl * a_sl, axis=0, keepdims=True)
        g_ao = jax.lax.dot(lhso_ref[...], rhs, preferred_element_type=_F32)
        d_ao = g_ao + n2q
        cm = d_ao[0:_SUB, :]
        for k in range(1, _NSL):
            cm = jnp.minimum(cm, d_ao[k * _SUB:(k + 1) * _SUB, :])
        mm = jnp.min(cm, axis=0, keepdims=True)
        ohc = (d_ao == mm).astype(_BF16)
        raw = jax.lax.dot(pay12_ref[...], ohc, preferred_element_type=_F32)
        nadv = (raw[0:3, :] + raw[3:6, :]) + raw[6:9, :]
        onenn = (raw[9:10, :] + raw[10:11, :]) + raw[11:12, :]
        g_aa = jax.lax.dot(lhsa_ref[...], rhs, preferred_element_type=_F32)
        d_aa = g_aa + n2q
        advk = _self_kappa_T(d_aa, pts9a_ref[...], a_sl, nadv)
        diff = advk - onenn
        part = jnp.reshape(jnp.sum(diff * diff), (1, 1))

        @pl.when(jnp.logical_and(b == 0, t == 0))
        def _():
            acc_ref[...] = jnp.zeros((1, 1), _F32)

        acc_ref[...] += part
        out_ref[...] = acc_ref[...] * (1.0 / (_B * _N))


def kernel(ori_data, adv_data, ori_normal):
    oriT = jnp.transpose(ori_data, (0, 2, 1))
    advT = jnp.transpose(adv_data, (0, 2, 1))
    row_spec = pl.BlockSpec((1, 3, _N), lambda b, p, t: (b, 0, 0))
    t_spec = pl.BlockSpec((1, _N, 3), lambda b, p, t: (b, 0, 0))
    out = pl.pallas_call(
        _curv_kernel,
        grid=(_B, 2, _NT),
        in_specs=[row_spec, row_spec, row_spec, t_spec, t_spec],
        out_specs=pl.BlockSpec((1, 1), lambda b, p, t: (0, 0)),
        out_shape=jax.ShapeDtypeStruct((1, 1), _F32),
        scratch_shapes=[
            pltpu.VMEM((1, _N), _F32),
            pltpu.VMEM((1, 1), _F32),
            pltpu.VMEM((_N, 6), _F32),
            pltpu.VMEM((_N, 6), _F32),
            pltpu.VMEM((9, _N), _BF16),
            pltpu.VMEM((9, _N), _BF16),
            pltpu.VMEM((12, _N), _BF16),
        ],
    )(ori_data, adv_data, ori_normal, oriT, advT)
    return out[0, 0]


# speedup vs baseline: 1.7828x; 1.2207x over previous
import jax
import jax.numpy as jnp
from jax.experimental import pallas as pl
from jax.experimental.pallas import tpu as pltpu

_B = 8
_N = 2048
_TQ = 256
_TPS = 2
_NT = _N // _TQ // _TPS
_SUB = 128
_NSL = _N // _SUB
_BIG = 1e30
_F32 = jnp.float32
_BF16 = jnp.bfloat16


def _split3_rows(x):
    hi = x.astype(_BF16).astype(_F32)
    r = x - hi
    mid = r.astype(_BF16).astype(_F32)
    lo = (r - mid).astype(_BF16).astype(_F32)
    return jnp.concatenate([hi, mid, lo], axis=0).astype(_BF16)


def _split3_cols_f32(x):
    hi = x.astype(_BF16).astype(_F32)
    r = x - hi
    mid = r.astype(_BF16).astype(_F32)
    lo = (r - mid).astype(_BF16).astype(_F32)
    return jnp.concatenate([hi, mid, lo], axis=1)


def _lane_top2_T(dist_t):
    m0 = dist_t[0:_SUB, :]
    m1 = jnp.full_like(m0, _BIG)
    for k in range(1, _NSL):
        x = dist_t[k * _SUB:(k + 1) * _SUB, :]
        lo = jnp.minimum(m0, x)
        hi = jnp.maximum(m0, x)
        m1 = jnp.minimum(m1, hi)
        m0 = lo
    return m0, m1


def _top3_vals_T(dist_t):
    m0, m1 = _lane_top2_T(dist_t)
    mm0 = jnp.min(m0, axis=0, keepdims=True)
    a0 = jnp.where(m0 == mm0, _BIG, m0)
    a1 = jnp.where(m1 == mm0, _BIG, m1)
    mm1 = jnp.min(jnp.minimum(a0, a1), axis=0, keepdims=True)
    b0 = jnp.where(a0 == mm1, _BIG, a0)
    b1 = jnp.where(a1 == mm1, _BIG, a1)
    mm2 = jnp.min(jnp.minimum(b0, b1), axis=0, keepdims=True)
    return mm1, mm2


def _self_kappa_T(dist_t, pts9_t, q_sl, nrm_sl):
    mm1, mm2 = _top3_vals_T(dist_t)
    oh1 = (dist_t == mm1).astype(_BF16)
    oh2 = (dist_t == mm2).astype(_BF16)
    r1 = jax.lax.dot(pts9_t, oh1, preferred_element_type=_F32)
    r2 = jax.lax.dot(pts9_t, oh2, preferred_element_type=_F32)

    def term(r):
        c = (r[0:3, :] + r[3:6, :]) + r[6:9, :]
        v = c - q_sl
        nv = jnp.sqrt(jnp.sum(v * v, axis=0, keepdims=True))
        s = jnp.sum(v * nrm_sl, axis=0, keepdims=True)
        return jnp.abs(s / jnp.maximum(nv, 1e-12))

    return 0.5 * (term(r1) + term(r2))


def _curv_kernel(ori_ref, adv_ref, nrm_ref, oriT_ref, advT_ref,
                 out_ref, kappa_ref, acc_ref,
                 lhso_ref, lhsa_ref, pts9o_ref, pts9a_ref, pay12_ref):
    b = pl.program_id(0)
    phase = pl.program_id(1)
    t = pl.program_id(2)
    ones3 = jnp.ones((3, _TQ), _F32)

    @pl.when(jnp.logical_and(phase == 0, t == 0))
    def _():
        ot = oriT_ref[0]
        at = advT_ref[0]
        n2o = jnp.sum(ot * ot, axis=1, keepdims=True)
        n2a = jnp.sum(at * at, axis=1, keepdims=True)
        lhso_ref[...] = jnp.concatenate([ot, _split3_cols_f32(n2o)], axis=1)
        lhsa_ref[...] = jnp.concatenate([at, _split3_cols_f32(n2a)], axis=1)
        pts9o_ref[...] = _split3_rows(ori_ref[0])
        pts9a_ref[...] = _split3_rows(adv_ref[0])

    @pl.when(phase == 0)
    def _():
        for u in range(_TPS):
            sl = pl.ds((t * _TPS + u) * _TQ, _TQ)
            o_sl = ori_ref[0, :, sl]
            n_sl = nrm_ref[0, :, sl]
            rhs = jnp.concatenate([o_sl * -2.0, ones3], axis=0)
            g = jax.lax.dot(lhso_ref[...], rhs, preferred_element_type=_F32)
            n2q = jnp.sum(o_sl * o_sl, axis=0, keepdims=True)
            dist_t = g + n2q
            kappa_ref[:, sl] = _self_kappa_T(dist_t, pts9o_ref[...],
                                             o_sl, n_sl)

    @pl.when(jnp.logical_and(phase == 1, t == 0))
    def _():
        pay12_ref[...] = jnp.concatenate(
            [_split3_rows(nrm_ref[0]), _split3_rows(kappa_ref[...])], axis=0)

    @pl.when(phase == 1)
    def _():
        parts = []
        for u in range(_TPS):
            sl = pl.ds((t * _TPS + u) * _TQ, _TQ)
            a_sl = adv_ref[0, :, sl]
            rhs = jnp.concatenate([a_sl * -2.0, ones3], axis=0)
            n2q = jnp.sum(a_sl * a_sl, axis=0, keepdims=True)
            g_ao = jax.lax.dot(lhso_ref[...], rhs, preferred_element_type=_F32)
            d_ao = g_ao + n2q
            cm = d_ao[0:_SUB, :]
            for k in range(1, _NSL):
                cm = jnp.minimum(cm, d_ao[k * _SUB:(k + 1) * _SUB, :])
            mm = jnp.min(cm, axis=0, keepdims=True)
            ohc = (d_ao == mm).astype(_BF16)
            raw = jax.lax.dot(pay12_ref[...], ohc, preferred_element_type=_F32)
            nadv = (raw[0:3, :] + raw[3:6, :]) + raw[6:9, :]
            onenn = (raw[9:10, :] + raw[10:11, :]) + raw[11:12, :]
            g_aa = jax.lax.dot(lhsa_ref[...], rhs, preferred_element_type=_F32)
            d_aa = g_aa + n2q
            advk = _self_kappa_T(d_aa, pts9a_ref[...], a_sl, nadv)
            diff = advk - onenn
            parts.append(jnp.sum(diff * diff))
        part = jnp.reshape(sum(parts), (1, 1))

        @pl.when(jnp.logical_and(b == 0, t == 0))
        def _():
            acc_ref[...] = jnp.zeros((1, 1), _F32)

        acc_ref[...] += part
        out_ref[...] = acc_ref[...] * (1.0 / (_B * _N))


def kernel(ori_data, adv_data, ori_normal):
    oriT = jnp.transpose(ori_data, (0, 2, 1))
    advT = jnp.transpose(adv_data, (0, 2, 1))
    row_spec = pl.BlockSpec((1, 3, _N), lambda b, p, t: (b, 0, 0))
    t_spec = pl.BlockSpec((1, _N, 3), lambda b, p, t: (b, 0, 0))
    out = pl.pallas_call(
        _curv_kernel,
        grid=(_B, 2, _NT),
        in_specs=[row_spec, row_spec, row_spec, t_spec, t_spec],
        out_specs=pl.BlockSpec((1, 1), lambda b, p, t: (0, 0)),
        out_shape=jax.ShapeDtypeStruct((1, 1), _F32),
        scratch_shapes=[
            pltpu.VMEM((1, _N), _F32),
            pltpu.VMEM((1, 1), _F32),
            pltpu.VMEM((_N, 6), _F32),
            pltpu.VMEM((_N, 6), _F32),
            pltpu.VMEM((9, _N), _BF16),
            pltpu.VMEM((9, _N), _BF16),
            pltpu.VMEM((12, _N), _BF16),
        ],
    )(ori_data, adv_data, ori_normal, oriT, advT)
    return out[0, 0]


# speedup vs baseline: 2.0328x; 1.1402x over previous
import jax
import jax.numpy as jnp
from jax.experimental import pallas as pl
from jax.experimental.pallas import tpu as pltpu

_B = 8
_N = 2048
_TQ = 256
_TPS = 4
_NT = _N // _TQ // _TPS
_SUB = 128
_NSL = _N // _SUB
_BIG = 1e30
_F32 = jnp.float32
_BF16 = jnp.bfloat16


def _split3_rows(x):
    hi = x.astype(_BF16).astype(_F32)
    r = x - hi
    mid = r.astype(_BF16).astype(_F32)
    lo = (r - mid).astype(_BF16).astype(_F32)
    return jnp.concatenate([hi, mid, lo], axis=0).astype(_BF16)


def _split3_cols_f32(x):
    hi = x.astype(_BF16).astype(_F32)
    r = x - hi
    mid = r.astype(_BF16).astype(_F32)
    lo = (r - mid).astype(_BF16).astype(_F32)
    return jnp.concatenate([hi, mid, lo], axis=1)


def _lane_top2_T(dist_t):
    m0 = dist_t[0:_SUB, :]
    m1 = jnp.full_like(m0, _BIG)
    for k in range(1, _NSL):
        x = dist_t[k * _SUB:(k + 1) * _SUB, :]
        lo = jnp.minimum(m0, x)
        hi = jnp.maximum(m0, x)
        m1 = jnp.minimum(m1, hi)
        m0 = lo
    return m0, m1


def _top3_vals_T(dist_t):
    m0, m1 = _lane_top2_T(dist_t)
    mm0 = jnp.min(m0, axis=0, keepdims=True)
    a0 = jnp.where(m0 == mm0, _BIG, m0)
    a1 = jnp.where(m1 == mm0, _BIG, m1)
    mm1 = jnp.min(jnp.minimum(a0, a1), axis=0, keepdims=True)
    b0 = jnp.where(a0 == mm1, _BIG, a0)
    b1 = jnp.where(a1 == mm1, _BIG, a1)
    mm2 = jnp.min(jnp.minimum(b0, b1), axis=0, keepdims=True)
    return mm1, mm2


def _self_kappa_T(dist_t, pts9_t, q_sl, nrm_sl):
    mm1, mm2 = _top3_vals_T(dist_t)
    oh1 = (dist_t == mm1).astype(_BF16)
    oh2 = (dist_t == mm2).astype(_BF16)
    r1 = jax.lax.dot(pts9_t, oh1, preferred_element_type=_F32)
    r2 = jax.lax.dot(pts9_t, oh2, preferred_element_type=_F32)

    def term(r):
        c = (r[0:3, :] + r[3:6, :]) + r[6:9, :]
        v = c - q_sl
        nv = jnp.sqrt(jnp.sum(v * v, axis=0, keepdims=True))
        s = jnp.sum(v * nrm_sl, axis=0, keepdims=True)
        return jnp.abs(s / jnp.maximum(nv, 1e-12))

    return 0.5 * (term(r1) + term(r2))


def _curv_kernel(ori_ref, adv_ref, nrm_ref, oriT_ref, advT_ref,
                 out_ref, kappa_ref, acc_ref,
                 lhso_ref, lhsa_ref, pts9o_ref, pts9a_ref, pay12_ref):
    b = pl.program_id(0)
    phase = pl.program_id(1)
    t = pl.program_id(2)
    ones3 = jnp.ones((3, _TQ), _F32)

    @pl.when(jnp.logical_and(phase == 0, t == 0))
    def _():
        ot = oriT_ref[0]
        at = advT_ref[0]
        n2o = jnp.sum(ot * ot, axis=1, keepdims=True)
        n2a = jnp.sum(at * at, axis=1, keepdims=True)
        lhso_ref[...] = jnp.concatenate([ot, _split3_cols_f32(n2o)], axis=1)
        lhsa_ref[...] = jnp.concatenate([at, _split3_cols_f32(n2a)], axis=1)
        pts9o_ref[...] = _split3_rows(ori_ref[0])
        pts9a_ref[...] = _split3_rows(adv_ref[0])

    @pl.when(phase == 0)
    def _():
        for u in range(_TPS):
            sl = pl.ds((t * _TPS + u) * _TQ, _TQ)
            o_sl = ori_ref[0, :, sl]
            n_sl = nrm_ref[0, :, sl]
            rhs = jnp.concatenate([o_sl * -2.0, ones3], axis=0)
            g = jax.lax.dot(lhso_ref[...], rhs, preferred_element_type=_F32)
            n2q = jnp.sum(o_sl * o_sl, axis=0, keepdims=True)
            dist_t = g + n2q
            kappa_ref[:, sl] = _self_kappa_T(dist_t, pts9o_ref[...],
                                             o_sl, n_sl)

    @pl.when(jnp.logical_and(phase == 1, t == 0))
    def _():
        pay12_ref[...] = jnp.concatenate(
            [_split3_rows(nrm_ref[0]), _split3_rows(kappa_ref[...])], axis=0)

    @pl.when(phase == 1)
    def _():
        parts = []
        for u in range(_TPS):
            sl = pl.ds((t * _TPS + u) * _TQ, _TQ)
            a_sl = adv_ref[0, :, sl]
            rhs = jnp.concatenate([a_sl * -2.0, ones3], axis=0)
            n2q = jnp.sum(a_sl * a_sl, axis=0, keepdims=True)
            g_ao = jax.lax.dot(lhso_ref[...], rhs, preferred_element_type=_F32)
            d_ao = g_ao + n2q
            cm = d_ao[0:_SUB, :]
            for k in range(1, _NSL):
                cm = jnp.minimum(cm, d_ao[k * _SUB:(k + 1) * _SUB, :])
            mm = jnp.min(cm, axis=0, keepdims=True)
            ohc = (d_ao == mm).astype(_BF16)
            raw = jax.lax.dot(pay12_ref[...], ohc, preferred_element_type=_F32)
            nadv = (raw[0:3, :] + raw[3:6, :]) + raw[6:9, :]
            onenn = (raw[9:10, :] + raw[10:11, :]) + raw[11:12, :]
            g_aa = jax.lax.dot(lhsa_ref[...], rhs, preferred_element_type=_F32)
            d_aa = g_aa + n2q
            advk = _self_kappa_T(d_aa, pts9a_ref[...], a_sl, nadv)
            diff = advk - onenn
            parts.append(jnp.sum(diff * diff))
        part = jnp.reshape(sum(parts), (1, 1))

        @pl.when(jnp.logical_and(b == 0, t == 0))
        def _():
            acc_ref[...] = jnp.zeros((1, 1), _F32)

        acc_ref[...] += part
        out_ref[...] = acc_ref[...] * (1.0 / (_B * _N))


def kernel(ori_data, adv_data, ori_normal):
    oriT = jnp.transpose(ori_data, (0, 2, 1))
    advT = jnp.transpose(adv_data, (0, 2, 1))
    row_spec = pl.BlockSpec((1, 3, _N), lambda b, p, t: (b, 0, 0))
    t_spec = pl.BlockSpec((1, _N, 3), lambda b, p, t: (b, 0, 0))
    out = pl.pallas_call(
        _curv_kernel,
        grid=(_B, 2, _NT),
        in_specs=[row_spec, row_spec, row_spec, t_spec, t_spec],
        out_specs=pl.BlockSpec((1, 1), lambda b, p, t: (0, 0)),
        out_shape=jax.ShapeDtypeStruct((1, 1), _F32),
        scratch_shapes=[
            pltpu.VMEM((1, _N), _F32),
            pltpu.VMEM((1, 1), _F32),
            pltpu.VMEM((_N, 6), _F32),
            pltpu.VMEM((_N, 6), _F32),
            pltpu.VMEM((9, _N), _BF16),
            pltpu.VMEM((9, _N), _BF16),
            pltpu.VMEM((12, _N), _BF16),
        ],
    )(ori_data, adv_data, ori_normal, oriT, advT)
    return out[0, 0]


# speedup vs baseline: 2.1900x; 1.0773x over previous
import jax
import jax.numpy as jnp
from jax.experimental import pallas as pl
from jax.experimental.pallas import tpu as pltpu

_B = 8
_N = 2048
_TQ = 256
_TPS = 8
_NT = _N // _TQ // _TPS
_SUB = 128
_NSL = _N // _SUB
_BIG = 1e30
_F32 = jnp.float32
_BF16 = jnp.bfloat16


def _split3_rows(x):
    hi = x.astype(_BF16).astype(_F32)
    r = x - hi
    mid = r.astype(_BF16).astype(_F32)
    lo = (r - mid).astype(_BF16).astype(_F32)
    return jnp.concatenate([hi, mid, lo], axis=0).astype(_BF16)


def _split3_cols_f32(x):
    hi = x.astype(_BF16).astype(_F32)
    r = x - hi
    mid = r.astype(_BF16).astype(_F32)
    lo = (r - mid).astype(_BF16).astype(_F32)
    return jnp.concatenate([hi, mid, lo], axis=1)


def _lane_top2_T(dist_t):
    m0 = dist_t[0:_SUB, :]
    m1 = jnp.full_like(m0, _BIG)
    for k in range(1, _NSL):
        x = dist_t[k * _SUB:(k + 1) * _SUB, :]
        lo = jnp.minimum(m0, x)
        hi = jnp.maximum(m0, x)
        m1 = jnp.minimum(m1, hi)
        m0 = lo
    return m0, m1


def _top3_vals_T(dist_t):
    m0, m1 = _lane_top2_T(dist_t)
    mm0 = jnp.min(m0, axis=0, keepdims=True)
    a0 = jnp.where(m0 == mm0, _BIG, m0)
    a1 = jnp.where(m1 == mm0, _BIG, m1)
    mm1 = jnp.min(jnp.minimum(a0, a1), axis=0, keepdims=True)
    b0 = jnp.where(a0 == mm1, _BIG, a0)
    b1 = jnp.where(a1 == mm1, _BIG, a1)
    mm2 = jnp.min(jnp.minimum(b0, b1), axis=0, keepdims=True)
    return mm1, mm2


def _self_kappa_T(dist_t, pts9_t, q_sl, nrm_sl):
    mm1, mm2 = _top3_vals_T(dist_t)
    oh1 = (dist_t == mm1).astype(_BF16)
    oh2 = (dist_t == mm2).astype(_BF16)
    r1 = jax.lax.dot(pts9_t, oh1, preferred_element_type=_F32)
    r2 = jax.lax.dot(pts9_t, oh2, preferred_element_type=_F32)

    def term(r):
        c = (r[0:3, :] + r[3:6, :]) + r[6:9, :]
        v = c - q_sl
        nv = jnp.sqrt(jnp.sum(v * v, axis=0, keepdims=True))
        s = jnp.sum(v * nrm_sl, axis=0, keepdims=True)
        return jnp.abs(s / jnp.maximum(nv, 1e-12))

    return 0.5 * (term(r1) + term(r2))


def _curv_kernel(ori_ref, adv_ref, nrm_ref, oriT_ref, advT_ref,
                 out_ref, kappa_ref, acc_ref,
                 lhso_ref, lhsa_ref, pts9o_ref, pts9a_ref, pay12_ref):
    b = pl.program_id(0)
    phase = pl.program_id(1)
    t = pl.program_id(2)
    ones3 = jnp.ones((3, _TQ), _F32)

    @pl.when(jnp.logical_and(phase == 0, t == 0))
    def _():
        ot = oriT_ref[0]
        at = advT_ref[0]
        n2o = jnp.sum(ot * ot, axis=1, keepdims=True)
        n2a = jnp.sum(at * at, axis=1, keepdims=True)
        lhso_ref[...] = jnp.concatenate([ot, _split3_cols_f32(n2o)], axis=1)
        lhsa_ref[...] = jnp.concatenate([at, _split3_cols_f32(n2a)], axis=1)
        pts9o_ref[...] = _split3_rows(ori_ref[0])
        pts9a_ref[...] = _split3_rows(adv_ref[0])

    @pl.when(phase == 0)
    def _():
        for u in range(_TPS):
            sl = pl.ds((t * _TPS + u) * _TQ, _TQ)
            o_sl = ori_ref[0, :, sl]
            n_sl = nrm_ref[0, :, sl]
            rhs = jnp.concatenate([o_sl * -2.0, ones3], axis=0)
            g = jax.lax.dot(lhso_ref[...], rhs, preferred_element_type=_F32)
            n2q = jnp.sum(o_sl * o_sl, axis=0, keepdims=True)
            dist_t = g + n2q
            kappa_ref[:, sl] = _self_kappa_T(dist_t, pts9o_ref[...],
                                             o_sl, n_sl)

    @pl.when(jnp.logical_and(phase == 1, t == 0))
    def _():
        pay12_ref[...] = jnp.concatenate(
            [_split3_rows(nrm_ref[0]), _split3_rows(kappa_ref[...])], axis=0)

    @pl.when(phase == 1)
    def _():
        parts = []
        for u in range(_TPS):
            sl = pl.ds((t * _TPS + u) * _TQ, _TQ)
            a_sl = adv_ref[0, :, sl]
            rhs = jnp.concatenate([a_sl * -2.0, ones3], axis=0)
            n2q = jnp.sum(a_sl * a_sl, axis=0, keepdims=True)
            g_ao = jax.lax.dot(lhso_ref[...], rhs, preferred_element_type=_F32)
            d_ao = g_ao + n2q
            cm = d_ao[0:_SUB, :]
            for k in range(1, _NSL):
                cm = jnp.minimum(cm, d_ao[k * _SUB:(k + 1) * _SUB, :])
            mm = jnp.min(cm, axis=0, keepdims=True)
            ohc = (d_ao == mm).astype(_BF16)
            raw = jax.lax.dot(pay12_ref[...], ohc, preferred_element_type=_F32)
            nadv = (raw[0:3, :] + raw[3:6, :]) + raw[6:9, :]
            onenn = (raw[9:10, :] + raw[10:11, :]) + raw[11:12, :]
            g_aa = jax.lax.dot(lhsa_ref[...], rhs, preferred_element_type=_F32)
            d_aa = g_aa + n2q
            advk = _self_kappa_T(d_aa, pts9a_ref[...], a_sl, nadv)
            diff = advk - onenn
            parts.append(jnp.sum(diff * diff))
        part = jnp.reshape(sum(parts), (1, 1))

        @pl.when(jnp.logical_and(b == 0, t == 0))
        def _():
            acc_ref[...] = jnp.zeros((1, 1), _F32)

        acc_ref[...] += part
        out_ref[...] = acc_ref[...] * (1.0 / (_B * _N))


def kernel(ori_data, adv_data, ori_normal):
    oriT = jnp.transpose(ori_data, (0, 2, 1))
    advT = jnp.transpose(adv_data, (0, 2, 1))
    row_spec = pl.BlockSpec((1, 3, _N), lambda b, p, t: (b, 0, 0))
    t_spec = pl.BlockSpec((1, _N, 3), lambda b, p, t: (b, 0, 0))
    out = pl.pallas_call(
        _curv_kernel,
        grid=(_B, 2, _NT),
        in_specs=[row_spec, row_spec, row_spec, t_spec, t_spec],
        out_specs=pl.BlockSpec((1, 1), lambda b, p, t: (0, 0)),
        out_shape=jax.ShapeDtypeStruct((1, 1), _F32),
        scratch_shapes=[
            pltpu.VMEM((1, _N), _F32),
            pltpu.VMEM((1, 1), _F32),
            pltpu.VMEM((_N, 6), _F32),
            pltpu.VMEM((_N, 6), _F32),
            pltpu.VMEM((9, _N), _BF16),
            pltpu.VMEM((9, _N), _BF16),
            pltpu.VMEM((12, _N), _BF16),
        ],
    )(ori_data, adv_data, ori_normal, oriT, advT)
    return out[0, 0]


# speedup vs baseline: 2.2273x; 1.0170x over previous
import jax
import jax.numpy as jnp
from jax.experimental import pallas as pl
from jax.experimental.pallas import tpu as pltpu

_B = 8
_N = 2048
_TQ = 256
_TPS = 8
_NT = _N // _TQ // _TPS
_SUB = 128
_NSL = _N // _SUB
_BIG = 1e30
_F32 = jnp.float32
_BF16 = jnp.bfloat16


def _split3_rows(x):
    hi = x.astype(_BF16).astype(_F32)
    r = x - hi
    mid = r.astype(_BF16).astype(_F32)
    lo = (r - mid).astype(_BF16).astype(_F32)
    return jnp.concatenate([hi, mid, lo], axis=0).astype(_BF16)


def _split3_cols_f32(x):
    hi = x.astype(_BF16).astype(_F32)
    r = x - hi
    mid = r.astype(_BF16).astype(_F32)
    lo = (r - mid).astype(_BF16).astype(_F32)
    return jnp.concatenate([hi, mid, lo], axis=1)


def _lane_top2_T(dist_t):
    m0 = dist_t[0:_SUB, :]
    m1 = jnp.full_like(m0, _BIG)
    for k in range(1, _NSL):
        x = dist_t[k * _SUB:(k + 1) * _SUB, :]
        lo = jnp.minimum(m0, x)
        hi = jnp.maximum(m0, x)
        m1 = jnp.minimum(m1, hi)
        m0 = lo
    return m0, m1


def _top3_vals_T(dist_t):
    m0, m1 = _lane_top2_T(dist_t)
    mm0 = jnp.min(m0, axis=0, keepdims=True)
    a0 = jnp.where(m0 == mm0, _BIG, m0)
    a1 = jnp.where(m1 == mm0, _BIG, m1)
    mm1 = jnp.min(jnp.minimum(a0, a1), axis=0, keepdims=True)
    b0 = jnp.where(a0 == mm1, _BIG, a0)
    b1 = jnp.where(a1 == mm1, _BIG, a1)
    mm2 = jnp.min(jnp.minimum(b0, b1), axis=0, keepdims=True)
    return mm1, mm2


def _self_kappa_T(dist_t, pts9_t, q_sl, nrm_sl):
    mm1, mm2 = _top3_vals_T(dist_t)
    oh1 = (dist_t == mm1).astype(_BF16)
    oh2 = (dist_t == mm2).astype(_BF16)
    r1 = jax.lax.dot(pts9_t, oh1, preferred_element_type=_F32)
    r2 = jax.lax.dot(pts9_t, oh2, preferred_element_type=_F32)

    def term(r):
        c = (r[0:3, :] + r[3:6, :]) + r[6:9, :]
        v = c - q_sl
        nv = jnp.sqrt(jnp.sum(v * v, axis=0, keepdims=True))
        s = jnp.sum(v * nrm_sl, axis=0, keepdims=True)
        return jnp.abs(s / jnp.maximum(nv, 1e-12))

    return 0.5 * (term(r1) + term(r2))


def _curv_kernel(ori_ref, adv_ref, nrm_ref, oriT_ref, advT_ref,
                 out_ref, kappa_ref, acc_ref,
                 lhso_ref, lhsa_ref, pts9o_ref, pts9a_ref, pay12_ref):
    b = pl.program_id(0)
    ones3 = jnp.ones((3, _TQ), _F32)

    ot = oriT_ref[0]
    at = advT_ref[0]
    n2o = jnp.sum(ot * ot, axis=1, keepdims=True)
    n2a = jnp.sum(at * at, axis=1, keepdims=True)
    lhso_ref[...] = jnp.concatenate([ot, _split3_cols_f32(n2o)], axis=1)
    lhsa_ref[...] = jnp.concatenate([at, _split3_cols_f32(n2a)], axis=1)
    pts9o_ref[...] = _split3_rows(ori_ref[0])
    pts9a_ref[...] = _split3_rows(adv_ref[0])

    for u in range(_N // _TQ):
        sl = pl.ds(u * _TQ, _TQ)
        o_sl = ori_ref[0, :, sl]
        n_sl = nrm_ref[0, :, sl]
        rhs = jnp.concatenate([o_sl * -2.0, ones3], axis=0)
        g = jax.lax.dot(lhso_ref[...], rhs, preferred_element_type=_F32)
        n2q = jnp.sum(o_sl * o_sl, axis=0, keepdims=True)
        dist_t = g + n2q
        kappa_ref[:, sl] = _self_kappa_T(dist_t, pts9o_ref[...], o_sl, n_sl)

    pay12_ref[...] = jnp.concatenate(
        [_split3_rows(nrm_ref[0]), _split3_rows(kappa_ref[...])], axis=0)

    parts = []
    for u in range(_N // _TQ):
        sl = pl.ds(u * _TQ, _TQ)
        a_sl = adv_ref[0, :, sl]
        rhs = jnp.concatenate([a_sl * -2.0, ones3], axis=0)
        n2q = jnp.sum(a_sl * a_sl, axis=0, keepdims=True)
        g_ao = jax.lax.dot(lhso_ref[...], rhs, preferred_element_type=_F32)
        d_ao = g_ao + n2q
        cm = d_ao[0:_SUB, :]
        for k in range(1, _NSL):
            cm = jnp.minimum(cm, d_ao[k * _SUB:(k + 1) * _SUB, :])
        mm = jnp.min(cm, axis=0, keepdims=True)
        ohc = (d_ao == mm).astype(_BF16)
        raw = jax.lax.dot(pay12_ref[...], ohc, preferred_element_type=_F32)
        nadv = (raw[0:3, :] + raw[3:6, :]) + raw[6:9, :]
        onenn = (raw[9:10, :] + raw[10:11, :]) + raw[11:12, :]
        g_aa = jax.lax.dot(lhsa_ref[...], rhs, preferred_element_type=_F32)
        d_aa = g_aa + n2q
        advk = _self_kappa_T(d_aa, pts9a_ref[...], a_sl, nadv)
        diff = advk - onenn
        parts.append(jnp.sum(diff * diff))
    part = jnp.reshape(sum(parts), (1, 1))

    @pl.when(b == 0)
    def _():
        acc_ref[...] = jnp.zeros((1, 1), _F32)

    acc_ref[...] += part
    out_ref[...] = acc_ref[...] * (1.0 / (_B * _N))


def kernel(ori_data, adv_data, ori_normal):
    oriT = jnp.transpose(ori_data, (0, 2, 1))
    advT = jnp.transpose(adv_data, (0, 2, 1))
    row_spec = pl.BlockSpec((1, 3, _N), lambda b: (b, 0, 0))
    t_spec = pl.BlockSpec((1, _N, 3), lambda b: (b, 0, 0))
    out = pl.pallas_call(
        _curv_kernel,
        grid=(_B,),
        in_specs=[row_spec, row_spec, row_spec, t_spec, t_spec],
        out_specs=pl.BlockSpec((1, 1), lambda b: (0, 0)),
        out_shape=jax.ShapeDtypeStruct((1, 1), _F32),
        scratch_shapes=[
            pltpu.VMEM((1, _N), _F32),
            pltpu.VMEM((1, 1), _F32),
            pltpu.VMEM((_N, 6), _F32),
            pltpu.VMEM((_N, 6), _F32),
            pltpu.VMEM((9, _N), _BF16),
            pltpu.VMEM((9, _N), _BF16),
            pltpu.VMEM((12, _N), _BF16),
        ],
    )(ori_data, adv_data, ori_normal, oriT, advT)
    return out[0, 0]
